# baseline (speedup 1.0000x reference)
import functools

import jax
import jax.numpy as jnp
from jax import lax
from jax.experimental import pallas as pl
from jax.experimental.pallas import tpu as pltpu
from jax.experimental.pallas import tpu_sc as plsc

TARGET_SAMPLES = 4096
LANES = 16

_info = plsc.get_sparse_core_info()
NC = _info.num_cores
NS = _info.num_subcores
NW = NC * NS


def _make_padder(total, batch, cu_pad):
    out_len = batch * TARGET_SAMPLES
    ch = out_len // NW
    wpr = TARGET_SAMPLES // ch
    flat_buf = total + TARGET_SAMPLES + LANES

    mesh = plsc.VectorSubcoreMesh(core_axis_name="c", subcore_axis_name="s")

    @functools.partial(
        pl.kernel,
        mesh=mesh,
        out_type=jax.ShapeDtypeStruct((out_len,), jnp.float32),
        scratch_types=[
            pltpu.VMEM((cu_pad,), jnp.int32),
            pltpu.VMEM((flat_buf,), jnp.float32),
            pltpu.VMEM((ch,), jnp.float32),
            pltpu.SemaphoreType.DMA,
            pltpu.SemaphoreType.DMA,
        ],
    )
    def padder(flat_hbm, cu_hbm, out_hbm, cu_v, flat_v, out_v, sem0, sem1):
        w = lax.axis_index("s") * NC + lax.axis_index("c")
        b = w // wpr
        i_start = (w % wpr) * ch

        cp0 = pltpu.async_copy(cu_hbm, cu_v, sem0)
        cp1 = pltpu.async_copy(flat_hbm, flat_v.at[pl.ds(0, total)], sem1)
        cp0.wait()
        cp1.wait()

        lanes = lax.iota(jnp.int32, LANES)
        cu_win = cu_v[pl.ds(b, LANES)]
        cu_b = cu_win[0]
        cu_b1 = cu_win[1]
        rel_len = cu_b1 - cu_b - i_start
        src = cu_b + i_start

        for j in range(ch // LANES):
            vals = flat_v[pl.ds(src + j * LANES, LANES)]
            ok = (lanes + (j * LANES)) < rel_len
            out_v[pl.ds(j * LANES, LANES)] = jnp.where(ok, vals, 0.0)

        pltpu.sync_copy(out_v, out_hbm.at[pl.ds(w * ch, ch)])

    return padder


def kernel(flat, cu_seqlens):
    total = flat.shape[0]
    batch = cu_seqlens.shape[0] - 1
    flat1 = flat.reshape(total)
    cu_pad = ((batch + 2 * LANES - 1) // LANES) * LANES
    cu_padded = jnp.pad(cu_seqlens, (0, cu_pad - cu_seqlens.shape[0]))
    out = _make_padder(total, batch, cu_pad)(flat1, cu_padded)
    return out.reshape(batch, TARGET_SAMPLES, 1)
